# baseline (speedup 1.0000x reference)
import functools

import jax
import jax.numpy as jnp
from jax import lax
from jax.experimental import pallas as pl
from jax.experimental.pallas import tpu as pltpu
from jax.experimental.pallas import tpu_sc as plsc

ROWS = 8192
COLS = 4096
NC = 2
NS = 16
NW = NC * NS
RPW = ROWS // NW
GROUPS = 8
BAND = 16 * GROUPS
NBANDS = RPW // BAND
CW = 256
NCHUNK = COLS // CW
PITCH = CW + 1


def _body(x_hbm, out_hbm, in_buf, out_buf):
    c = lax.axis_index("c")
    s = lax.axis_index("s")
    wid = c * NS + s

    iota = lax.iota(jnp.int32, 16)
    row_idx = [iota + 16 * g for g in range(GROUPS)]

    for band in range(NBANDS):
        r0 = wid * RPW + band * BAND
        accs = [jnp.zeros((16,), jnp.float32) for _ in range(GROUPS)]
        for chunk in range(NCHUNK):
            c0 = chunk * CW
            pltpu.sync_copy(
                x_hbm.at[pl.ds(r0, BAND), pl.ds(c0, CW)],
                in_buf.at[:, pl.ds(0, CW)],
            )

            def col_step(j, accs):
                cj = jnp.broadcast_to(j, (16,)).astype(jnp.int32)
                new = []
                for g in range(GROUPS):
                    v = plsc.load_gather(in_buf, [row_idx[g], cj])
                    a = accs[g] + v
                    plsc.store_scatter(out_buf, [row_idx[g], cj], a)
                    new.append(a)
                return tuple(new)

            accs = list(lax.fori_loop(0, CW, col_step, tuple(accs)))

            pltpu.sync_copy(
                out_buf.at[:, pl.ds(0, CW)],
                out_hbm.at[pl.ds(r0, BAND), pl.ds(c0, CW)],
            )


def kernel(x):
    mesh = plsc.VectorSubcoreMesh(core_axis_name="c", subcore_axis_name="s")
    run = pl.kernel(
        _body,
        out_type=jax.ShapeDtypeStruct((ROWS, COLS), jnp.float32),
        mesh=mesh,
        scratch_types=[
            pltpu.VMEM((BAND, PITCH), jnp.float32),
            pltpu.VMEM((BAND, PITCH), jnp.float32),
        ],
        compiler_params=pltpu.CompilerParams(
            use_tc_tiling_on_sc=False, needs_layout_passes=False
        ),
    )
    return run(x)


# speedup vs baseline: 1.4151x; 1.4151x over previous
import functools

import jax
import jax.numpy as jnp
from jax import lax
from jax.experimental import pallas as pl
from jax.experimental.pallas import tpu as pltpu
from jax.experimental.pallas import tpu_sc as plsc

ROWS = 8192
COLS = 4096
NC = 2
NS = 16
NW = NC * NS
RPW = ROWS // NW
GROUPS = 8
BAND = 16 * GROUPS
NBANDS = RPW // BAND
CW = 256
NCHUNK = COLS // CW
PITCH = CW + 1


def _body(x_hbm, out_hbm, in_buf, out_buf):
    c = lax.axis_index("c")
    s = lax.axis_index("s")
    wid = c * NS + s

    iota = lax.iota(jnp.int32, 16)
    row_idx = [iota + 16 * g for g in range(GROUPS)]

    for band in range(NBANDS):
        r0 = wid * RPW + band * BAND
        accs0 = tuple(jnp.zeros((16,), jnp.float32) for _ in range(GROUPS))

        def chunk_step(chunk, accs, r0=r0):
            c0 = chunk * CW
            pltpu.sync_copy(
                x_hbm.at[pl.ds(r0, BAND), pl.ds(c0, CW)],
                in_buf.at[:, pl.ds(0, CW)],
            )

            @plsc.parallel_loop(0, CW, step=1, unroll=4, carry=accs)
            def col_step(j, accs):
                cj = jnp.broadcast_to(j, (16,)).astype(jnp.int32)
                vs = [
                    plsc.load_gather(in_buf, [row_idx[g], cj])
                    for g in range(GROUPS)
                ]
                new = [accs[g] + vs[g] for g in range(GROUPS)]
                for g in range(GROUPS):
                    plsc.store_scatter(out_buf, [row_idx[g], cj], new[g])
                return tuple(new)

            pltpu.sync_copy(
                out_buf.at[:, pl.ds(0, CW)],
                out_hbm.at[pl.ds(r0, BAND), pl.ds(c0, CW)],
            )
            return col_step

        lax.fori_loop(0, NCHUNK, chunk_step, accs0)


def kernel(x):
    mesh = plsc.VectorSubcoreMesh(core_axis_name="c", subcore_axis_name="s")
    run = pl.kernel(
        _body,
        out_type=jax.ShapeDtypeStruct((ROWS, COLS), jnp.float32),
        mesh=mesh,
        scratch_types=[
            pltpu.VMEM((BAND, PITCH), jnp.float32),
            pltpu.VMEM((BAND, PITCH), jnp.float32),
        ],
        compiler_params=pltpu.CompilerParams(
            use_tc_tiling_on_sc=False, needs_layout_passes=False
        ),
    )
    return run(x)
